# baseline (speedup 1.0000x reference)
import jax
import jax.numpy as jnp
from jax import lax
from jax.experimental import pallas as pl
from jax.experimental.pallas import tpu as pltpu
from jax.experimental.pallas import tpu_sc as plsc

EMB_DIM = 64
BATCH = 16384
NUM_CORES = 2
NUM_SUBCORES = 16
NW = NUM_CORES * NUM_SUBCORES
B_W = BATCH // NW
LANES = 16
D_CH = EMB_DIM // LANES
GROUP = 32
NG = B_W // GROUP
SLAB = 8


def _bpr_body(user_hbm, item_i_hbm, item_j_hbm, tab_u, tab_it,
              out_i_hbm, out_j_hbm,
              idx_u, idx_i, idx_j, u_s, vi_s, vj_s,
              out_i_v, out_j_v, sem0, sem1):
    wid = lax.axis_index("s") * NUM_CORES + lax.axis_index("c")
    base = wid * B_W
    sems = (sem0, sem1)

    pltpu.sync_copy(user_hbm.at[wid], idx_u)
    pltpu.sync_copy(item_i_hbm.at[wid], idx_i)
    pltpu.sync_copy(item_j_hbm.at[wid], idx_j)

    lane_iota = lax.iota(jnp.int32, LANES)
    perms = [lane_iota ^ s for s in (8, 4, 2, 1)]

    def lane_sum(v):
        for p in perms:
            v = v + v[p]
        return v

    def load_ivs(g0):
        out = []
        for b in range(GROUP // LANES):
            sl = pl.ds(g0 + b * LANES, LANES)
            out.append((idx_u[sl], idx_i[sl], idx_j[sl]))
        return out

    def fire(g, slot):
        g0 = g * GROUP
        for b, (iu, ii, ij) in enumerate(load_ivs(g0)):
            bu, bi, bj = iu >> 3, ii >> 3, ij >> 3
            qu, qi, qj = iu & 7, ii & 7, ij & 7
            for k in range(LANES):
                kk = b * LANES + k
                for bv, qv, tab, slab in (
                        (bu, qu, tab_u, u_s), (bi, qi, tab_it, vi_s),
                        (bj, qj, tab_it, vj_s)):
                    pltpu.async_copy(tab.at[bv[k], qv[k]],
                                     slab.at[slot, kk // SLAB, kk % SLAB],
                                     sems[slot])

    def drain(slot):
        for slab in (u_s, vi_s, vj_s):
            for j in range(GROUP // SLAB):
                pltpu.make_async_copy(tab_u.at[0], slab.at[slot, j],
                                      sems[slot]).wait()

    def compute(g, slot):
        g0 = g * GROUP
        for b in range(GROUP // LANES):
            res_i = jnp.zeros((LANES,), jnp.float32)
            res_j = jnp.zeros((LANES,), jnp.float32)
            for k in range(LANES):
                kk = b * LANES + k
                jj, qq = kk // SLAB, kk % SLAB
                u = u_s[slot, jj, qq, pl.ds(0, LANES)]
                acc_i = u * vi_s[slot, jj, qq, pl.ds(0, LANES)]
                acc_j = u * vj_s[slot, jj, qq, pl.ds(0, LANES)]
                for c in range(1, D_CH):
                    sl = pl.ds(c * LANES, LANES)
                    u = u_s[slot, jj, qq, sl]
                    acc_i = acc_i + u * vi_s[slot, jj, qq, sl]
                    acc_j = acc_j + u * vj_s[slot, jj, qq, sl]
                res_i = jnp.where(lane_iota == k, lane_sum(acc_i), res_i)
                res_j = jnp.where(lane_iota == k, lane_sum(acc_j), res_j)
            out_i_v[pl.ds(g0 + b * LANES, LANES)] = res_i
            out_j_v[pl.ds(g0 + b * LANES, LANES)] = res_j

    fire(0, 0)

    def super_body(h, carry):
        g0 = 2 * h
        fire(g0 + 1, 1)
        drain(0)
        compute(g0, 0)

        @pl.when(h + 1 < NG // 2)
        def _():
            fire(g0 + 2, 0)

        drain(1)
        compute(g0 + 1, 1)
        return carry

    lax.fori_loop(0, NG // 2, super_body, 0)

    pltpu.sync_copy(out_i_v, out_i_hbm.at[pl.ds(base, B_W)])
    pltpu.sync_copy(out_j_v, out_j_hbm.at[pl.ds(base, B_W)])


@jax.jit
def _bpr(user_r, item_i_r, item_j_r, user_embs, item_embs):
    mesh = plsc.VectorSubcoreMesh(core_axis_name="c", subcore_axis_name="s")
    f = pl.kernel(
        _bpr_body,
        mesh=mesh,
        out_type=[
            jax.ShapeDtypeStruct((BATCH,), jnp.float32),
            jax.ShapeDtypeStruct((BATCH,), jnp.float32),
        ],
        scratch_types=[
            pltpu.VMEM((B_W,), jnp.int32),
            pltpu.VMEM((B_W,), jnp.int32),
            pltpu.VMEM((B_W,), jnp.int32),
            pltpu.VMEM((2, GROUP // SLAB, SLAB, EMB_DIM), jnp.float32),
            pltpu.VMEM((2, GROUP // SLAB, SLAB, EMB_DIM), jnp.float32),
            pltpu.VMEM((2, GROUP // SLAB, SLAB, EMB_DIM), jnp.float32),
            pltpu.VMEM((B_W,), jnp.float32),
            pltpu.VMEM((B_W,), jnp.float32),
            pltpu.SemaphoreType.DMA,
            pltpu.SemaphoreType.DMA,
        ],
    )
    out_i, out_j = f(user_r, item_i_r, item_j_r, user_embs, item_embs)
    return out_i, out_j


def kernel(user, item_i, item_j, user_embs, item_embs):
    user_r = user.reshape(NW, B_W)
    item_i_r = item_i.reshape(NW, B_W)
    item_j_r = item_j.reshape(NW, B_W)
    tab_u = user_embs.reshape(user_embs.shape[0] // SLAB, SLAB, EMB_DIM)
    tab_it = item_embs.reshape(item_embs.shape[0] // SLAB, SLAB, EMB_DIM)
    return _bpr(user_r, item_i_r, item_j_r, tab_u, tab_it)


# speedup vs baseline: 1.0194x; 1.0194x over previous
import jax
import jax.numpy as jnp
from jax import lax
from jax.experimental import pallas as pl
from jax.experimental.pallas import tpu as pltpu
from jax.experimental.pallas import tpu_sc as plsc

EMB_DIM = 64
BATCH = 16384
NUM_CORES = 2
NUM_SUBCORES = 16
NW = NUM_CORES * NUM_SUBCORES
B_W = BATCH // NW
LANES = 16
D_CH = EMB_DIM // LANES
GROUP = 16
NG = B_W // GROUP
SLAB = 8


def _bpr_body(user_hbm, item_i_hbm, item_j_hbm, tab_u, tab_it,
              out_i_hbm, out_j_hbm,
              idx_u, idx_i, idx_j, u_s, vi_s, vj_s,
              out_i_v, out_j_v, sem0, sem1):
    wid = lax.axis_index("s") * NUM_CORES + lax.axis_index("c")
    base = wid * B_W
    sems = (sem0, sem1)

    pltpu.sync_copy(user_hbm.at[wid], idx_u)
    pltpu.sync_copy(item_i_hbm.at[wid], idx_i)
    pltpu.sync_copy(item_j_hbm.at[wid], idx_j)

    lane_iota = lax.iota(jnp.int32, LANES)
    perms = [lane_iota ^ s for s in (8, 4, 2, 1)]

    def lane_sum(v):
        for p in perms:
            v = v + v[p]
        return v

    def load_ivs(g0):
        out = []
        for b in range(GROUP // LANES):
            sl = pl.ds(g0 + b * LANES, LANES)
            out.append((idx_u[sl], idx_i[sl], idx_j[sl]))
        return out

    def fire(g, slot):
        g0 = g * GROUP
        for b, (iu, ii, ij) in enumerate(load_ivs(g0)):
            bu, bi, bj = iu >> 3, ii >> 3, ij >> 3
            qu, qi, qj = iu & 7, ii & 7, ij & 7
            for k in range(LANES):
                kk = b * LANES + k
                for bv, qv, tab, slab in (
                        (bu, qu, tab_u, u_s), (bi, qi, tab_it, vi_s),
                        (bj, qj, tab_it, vj_s)):
                    pltpu.async_copy(tab.at[bv[k], qv[k]],
                                     slab.at[slot, kk // SLAB, kk % SLAB],
                                     sems[slot])

    def drain(slot):
        for slab in (u_s, vi_s, vj_s):
            for j in range(GROUP // SLAB):
                pltpu.make_async_copy(tab_u.at[0], slab.at[slot, j],
                                      sems[slot]).wait()

    def compute(g, slot):
        g0 = g * GROUP
        for b in range(GROUP // LANES):
            res_i = jnp.zeros((LANES,), jnp.float32)
            res_j = jnp.zeros((LANES,), jnp.float32)
            for k in range(LANES):
                kk = b * LANES + k
                jj, qq = kk // SLAB, kk % SLAB
                u = u_s[slot, jj, qq, pl.ds(0, LANES)]
                acc_i = u * vi_s[slot, jj, qq, pl.ds(0, LANES)]
                acc_j = u * vj_s[slot, jj, qq, pl.ds(0, LANES)]
                for c in range(1, D_CH):
                    sl = pl.ds(c * LANES, LANES)
                    u = u_s[slot, jj, qq, sl]
                    acc_i = acc_i + u * vi_s[slot, jj, qq, sl]
                    acc_j = acc_j + u * vj_s[slot, jj, qq, sl]
                res_i = jnp.where(lane_iota == k, lane_sum(acc_i), res_i)
                res_j = jnp.where(lane_iota == k, lane_sum(acc_j), res_j)
            out_i_v[pl.ds(g0 + b * LANES, LANES)] = res_i
            out_j_v[pl.ds(g0 + b * LANES, LANES)] = res_j

    fire(0, 0)

    def super_body(h, carry):
        g0 = 2 * h
        fire(g0 + 1, 1)
        drain(0)
        compute(g0, 0)

        @pl.when(h + 1 < NG // 2)
        def _():
            fire(g0 + 2, 0)

        drain(1)
        compute(g0 + 1, 1)
        return carry

    lax.fori_loop(0, NG // 2, super_body, 0)

    pltpu.sync_copy(out_i_v, out_i_hbm.at[pl.ds(base, B_W)])
    pltpu.sync_copy(out_j_v, out_j_hbm.at[pl.ds(base, B_W)])


@jax.jit
def _bpr(user_r, item_i_r, item_j_r, user_embs, item_embs):
    mesh = plsc.VectorSubcoreMesh(core_axis_name="c", subcore_axis_name="s")
    f = pl.kernel(
        _bpr_body,
        mesh=mesh,
        out_type=[
            jax.ShapeDtypeStruct((BATCH,), jnp.float32),
            jax.ShapeDtypeStruct((BATCH,), jnp.float32),
        ],
        scratch_types=[
            pltpu.VMEM((B_W,), jnp.int32),
            pltpu.VMEM((B_W,), jnp.int32),
            pltpu.VMEM((B_W,), jnp.int32),
            pltpu.VMEM((2, GROUP // SLAB, SLAB, EMB_DIM), jnp.float32),
            pltpu.VMEM((2, GROUP // SLAB, SLAB, EMB_DIM), jnp.float32),
            pltpu.VMEM((2, GROUP // SLAB, SLAB, EMB_DIM), jnp.float32),
            pltpu.VMEM((B_W,), jnp.float32),
            pltpu.VMEM((B_W,), jnp.float32),
            pltpu.SemaphoreType.DMA,
            pltpu.SemaphoreType.DMA,
        ],
    )
    out_i, out_j = f(user_r, item_i_r, item_j_r, user_embs, item_embs)
    return out_i, out_j


def kernel(user, item_i, item_j, user_embs, item_embs):
    user_r = user.reshape(NW, B_W)
    item_i_r = item_i.reshape(NW, B_W)
    item_j_r = item_j.reshape(NW, B_W)
    tab_u = user_embs.reshape(user_embs.shape[0] // SLAB, SLAB, EMB_DIM)
    tab_it = item_embs.reshape(item_embs.shape[0] // SLAB, SLAB, EMB_DIM)
    return _bpr(user_r, item_i_r, item_j_r, tab_u, tab_it)
